# baseline (speedup 1.0000x reference)
import functools

import numpy as np
import jax
from jax import lax
from jax.experimental import pallas as pl
from jax.experimental.pallas import tpu as pltpu
from jax.experimental.pallas import tpu_sc as plsc


def kernel(frames):
    C, T, H, W = frames.shape
    S = T // 4
    info = plsc.get_sparse_core_info()
    NC, NS = info.num_cores, info.num_subcores
    NW = NC * NS
    RPW = (C * T) // NW

    mesh = plsc.VectorSubcoreMesh(core_axis_name="c", subcore_axis_name="s")

    @functools.partial(
        pl.kernel,
        out_type=[
            jax.ShapeDtypeStruct((C, S, H, W), frames.dtype),
            jax.ShapeDtypeStruct((C, T, H, W), frames.dtype),
        ],
        mesh=mesh,
        scratch_types=[
            pltpu.VMEM((2, H, W), frames.dtype),
            pltpu.SemaphoreType.DMA((2,)),
            pltpu.SemaphoreType.DMA((2,)),
        ],
    )
    def sc_pack(x, slow, fast, buf, in_sem, out_sem):
        w = lax.axis_index("s") * NC + lax.axis_index("c")

        def in_copy(i, b):
            r = w * RPW + i
            return pltpu.make_async_copy(
                x.at[r // T, r % T], buf.at[b], in_sem.at[b])

        def out_start(i, b):
            r = w * RPW + i
            c, t = r // T, r % T
            pltpu.make_async_copy(buf.at[b], fast.at[c, t],
                                  out_sem.at[b]).start()
            j = ((S - 1) * t + (T - 2)) // (T - 1)
            hit = (j * (T - 1)) // (S - 1) == t

            @pl.when(hit)
            def _():
                pltpu.make_async_copy(buf.at[b], slow.at[c, j],
                                      out_sem.at[b]).start()

        def out_wait(i, b):
            r = w * RPW + i
            c, t = r // T, r % T
            pltpu.make_async_copy(buf.at[b], fast.at[c, t],
                                  out_sem.at[b]).wait()
            j = ((S - 1) * t + (T - 2)) // (T - 1)
            hit = (j * (T - 1)) // (S - 1) == t

            @pl.when(hit)
            def _():
                pltpu.make_async_copy(buf.at[b], slow.at[c, j],
                                      out_sem.at[b]).wait()

        in_copy(0, 0).start()
        for i in range(RPW):
            b = i % 2
            if i + 1 < RPW:
                if i - 1 >= 0:
                    out_wait(i - 1, 1 - b)
                in_copy(i + 1, 1 - b).start()
            in_copy(i, b).wait()
            out_start(i, b)
        for i in (RPW - 2, RPW - 1):
            out_wait(i, i % 2)

    slow, fast = sc_pack(frames)
    return (slow, fast)
